# baseline (speedup 1.0000x reference)
import functools

import jax
import jax.numpy as jnp
from jax import lax
from jax.experimental import pallas as pl
from jax.experimental.pallas import tpu as pltpu
from jax.experimental.pallas import tpu_sc as plsc

SEQ_LEN = 8192
NUM_CORES = 2
NUM_SUBCORES = 16
LANES = 16
NUM_WORKERS = NUM_CORES * NUM_SUBCORES
ROWS_PER_WORKER = SEQ_LEN // NUM_WORKERS
CHUNKS = ROWS_PER_WORKER // LANES


def _gate_body(gs_hbm, out0_hbm, out1_hbm, gs_v, o0_v, o1_v):
    wid = lax.axis_index("s") * NUM_CORES + lax.axis_index("c")
    base = wid * ROWS_PER_WORKER

    pltpu.sync_copy(gs_hbm.at[pl.ds(base * 2, ROWS_PER_WORKER * 2)], gs_v)

    for c in range(CHUNKS):
        even = c * 2 * LANES + 2 * lax.iota(jnp.int32, LANES)
        g0 = plsc.load_gather(gs_v, [even])
        g1 = plsc.load_gather(gs_v, [even + 1])
        m = jnp.maximum(g0, g1)
        e0 = jnp.exp(g0 - m)
        e1 = jnp.exp(g1 - m)
        denom = e0 + e1
        s0 = e0 / denom
        s1 = e1 / denom
        sel = g0 >= g1
        h0 = jnp.where(sel, 1.0, 0.0)
        h1 = jnp.where(sel, 0.0, 1.0)
        o0_v[pl.ds(c * LANES, LANES)] = h0 - s0 + s0
        o1_v[pl.ds(c * LANES, LANES)] = h1 - s1 + s1

    pltpu.sync_copy(o0_v, out0_hbm.at[pl.ds(base, ROWS_PER_WORKER)])
    pltpu.sync_copy(o1_v, out1_hbm.at[pl.ds(base, ROWS_PER_WORKER)])


@functools.cache
def _gate_kernel():
    return pl.kernel(
        _gate_body,
        out_type=(
            jax.ShapeDtypeStruct((SEQ_LEN,), jnp.float32),
            jax.ShapeDtypeStruct((SEQ_LEN,), jnp.float32),
        ),
        mesh=plsc.VectorSubcoreMesh(
            core_axis_name="c", subcore_axis_name="s",
            num_cores=NUM_CORES, num_subcores=NUM_SUBCORES,
        ),
        scratch_types=[
            pltpu.VMEM((2 * ROWS_PER_WORKER,), jnp.float32),
            pltpu.VMEM((ROWS_PER_WORKER,), jnp.float32),
            pltpu.VMEM((ROWS_PER_WORKER,), jnp.float32),
        ],
        compiler_params=pltpu.CompilerParams(needs_layout_passes=False),
    )


def kernel(x, W, global_gate_score, smoothing_factor):
    del x, W, smoothing_factor
    gs_flat = global_gate_score.reshape(-1)
    return _gate_kernel()(gs_flat)


# speedup vs baseline: 2.2514x; 2.2514x over previous
import jax
import jax.numpy as jnp
from jax import lax
from jax.experimental import pallas as pl
from jax.experimental.pallas import tpu as pltpu

SEQ_LEN = 8192
ROWS = 128
COLS = 128
PAIRS = 64


def _gate_body(gs_ref, out0_ref, out1_ref):
    a = gs_ref[...]
    nxt = pltpu.roll(a, COLS - 1, 1)
    prv = pltpu.roll(a, 1, 1)
    lane = lax.broadcasted_iota(jnp.int32, (ROWS, COLS), 1)
    is_even = lane % 2 == 0
    partner = jnp.where(is_even, nxt, prv)
    m = jnp.maximum(a, partner)
    e_own = jnp.exp(a - m)
    e_par = jnp.exp(partner - m)
    s = e_own / (e_own + e_par)
    ge_f = jnp.where(a >= partner, 1.0, 0.0)
    gt_f = jnp.where(a > partner, 1.0, 0.0)
    r = jnp.where(is_even, ge_f, gt_f) - s + s
    j = lax.broadcasted_iota(jnp.int32, (COLS, PAIRS), 0)
    k = lax.broadcasted_iota(jnp.int32, (COLS, PAIRS), 1)
    dn = (((1,), (0,)), ((), ()))
    out0_ref[...] = lax.dot_general(
        r, (j == 2 * k).astype(jnp.float32), dn,
        precision=lax.Precision.HIGHEST, preferred_element_type=jnp.float32)
    out1_ref[...] = lax.dot_general(
        r, (j == 2 * k + 1).astype(jnp.float32), dn,
        precision=lax.Precision.HIGHEST, preferred_element_type=jnp.float32)


def kernel(x, W, global_gate_score, smoothing_factor):
    del x, W, smoothing_factor
    gs = global_gate_score.reshape(ROWS, COLS)
    out0, out1 = pl.pallas_call(
        _gate_body,
        out_shape=(
            jax.ShapeDtypeStruct((ROWS, PAIRS), jnp.float32),
            jax.ShapeDtypeStruct((ROWS, PAIRS), jnp.float32),
        ),
    )(gs)
    return out0.reshape(SEQ_LEN), out1.reshape(SEQ_LEN)


# speedup vs baseline: 11.6654x; 5.1815x over previous
import jax
import jax.numpy as jnp
from jax.experimental import pallas as pl

SEQ_LEN = 8192


def _gate_body(gs_ref, out0_ref, out1_ref):
    g0 = gs_ref[0:1, :]
    g1 = gs_ref[1:2, :]
    m = jnp.maximum(g0, g1)
    e0 = jnp.exp(g0 - m)
    e1 = jnp.exp(g1 - m)
    denom = e0 + e1
    s0 = e0 / denom
    s1 = e1 / denom
    sel = g0 >= g1
    out0_ref[...] = jnp.where(sel, 1.0, 0.0) - s0 + s0
    out1_ref[...] = jnp.where(sel, 0.0, 1.0) - s1 + s1


def kernel(x, W, global_gate_score, smoothing_factor):
    del x, W, smoothing_factor
    gt = global_gate_score.T
    out0, out1 = pl.pallas_call(
        _gate_body,
        out_shape=(
            jax.ShapeDtypeStruct((1, SEQ_LEN), jnp.float32),
            jax.ShapeDtypeStruct((1, SEQ_LEN), jnp.float32),
        ),
    )(gt)
    return out0.reshape(SEQ_LEN), out1.reshape(SEQ_LEN)
